# baseline (speedup 1.0000x reference)
import functools

import jax
import jax.numpy as jnp
import numpy as np
from jax import lax
from jax.experimental import pallas as pl
from jax.experimental.pallas import tpu as pltpu
from jax.experimental.pallas import tpu_sc as plsc

VOCAB = 100000
D_MODEL = 768
MAX_LEN = 4096
LANES = 16

NUM_CORES = 2
NUM_SUBCORES = 16
NW = NUM_CORES * NUM_SUBCORES

CP = 16


@functools.lru_cache(maxsize=None)
def _trig_tables(seq, d_model):
    i2 = np.arange(0, d_model, 2, dtype=np.float32)
    div = np.power(np.float32(10000.0), (i2 / np.float32(d_model)).astype(np.float32))
    divd = np.repeat(div, 2)

    jbase = (np.arange(seq // CP, dtype=np.float32) * np.float32(CP))[:, None]
    a_ang = (jbase / divd[None, :]).astype(np.float32)
    a_tab = np.concatenate([np.sin(a_ang), np.cos(a_ang)], axis=1)

    r = np.arange(CP, dtype=np.float32)[:, None]
    b_ang = (r / divd[None, :]).astype(np.float32)
    b_tab = np.concatenate([np.sin(b_ang), np.cos(b_ang)], axis=1)

    return jnp.asarray(a_tab.astype(np.float32)), jnp.asarray(b_tab.astype(np.float32))


def _make_sc_kernel(batch, seq, d_model):
    ppw = seq // NW
    nch = ppw // CP
    rows = batch * CP
    mesh = plsc.VectorSubcoreMesh(core_axis_name="c", subcore_axis_name="s")

    @functools.partial(
        pl.kernel,
        out_type=jax.ShapeDtypeStruct((batch * seq, d_model), jnp.float32),
        mesh=mesh,
        scratch_types=[
            pltpu.VMEM((batch, ppw), jnp.int32),
            pltpu.VMEM((rows, d_model), jnp.float32),
            pltpu.VMEM((rows, d_model), jnp.float32),
            pltpu.VMEM((2 * d_model,), jnp.float32),
            pltpu.VMEM((2 * d_model,), jnp.float32),
            pltpu.VMEM((CP, 2 * d_model), jnp.float32),
            pltpu.SemaphoreType.DMA,
            pltpu.SemaphoreType.DMA,
            pltpu.SemaphoreType.DMA,
            pltpu.SemaphoreType.DMA,
            pltpu.SemaphoreType.DMA,
            pltpu.SemaphoreType.DMA,
            pltpu.SemaphoreType.DMA,
        ],
    )
    def emb_kernel(x_hbm, table_hbm, a_hbm, b_hbm, out_hbm,
                   idx_v, rows0, rows1, a0, a1, b_v,
                   sem_i, sem_g0, sem_g1, sem_a0, sem_a1, sem_o0, sem_o1):
        rows_b = (rows0, rows1)
        a_b = (a0, a1)
        sem_g = (sem_g0, sem_g1)
        sem_a = (sem_a0, sem_a1)
        sem_o = (sem_o0, sem_o1)

        wid = lax.axis_index("s") * NUM_CORES + lax.axis_index("c")
        q0 = wid * ppw
        j0 = wid * nch

        startup = [
            pltpu.async_copy(x_hbm.at[b, pl.ds(q0, ppw)], idx_v.at[b], sem_i)
            for b in range(batch)
        ]
        startup.append(pltpu.async_copy(b_hbm, b_v, sem_i))
        for c in startup:
            c.wait()

        lane = lax.iota(jnp.int32, 16)
        even = (lane % 2) == 0

        def start(j):
            s = j % 2
            gs = [
                pltpu.async_copy(
                    table_hbm.at[idx_v.at[b, pl.ds(j * CP, CP)]],
                    rows_b[s].at[pl.ds(b * CP, CP)],
                    sem_g[s],
                )
                for b in range(batch)
            ]
            gs.append(pltpu.async_copy(a_hbm.at[j0 + j], a_b[s], sem_a[s]))
            return gs

        started = {0: start(0)}
        pending_out = {}
        for j in range(nch):
            s = j % 2
            if j + 1 < nch:
                if j - 1 >= 0:
                    for c in pending_out[j - 1]:
                        c.wait()
                started[j + 1] = start(j + 1)
            for c in started[j]:
                c.wait()

            def row_body(r, carry, s=s):
                for g in range(d_model // LANES):
                    sl = pl.ds(g * LANES, LANES)
                    sl2 = pl.ds(d_model + g * LANES, LANES)
                    a_s = a_b[s][sl]
                    a_c = a_b[s][sl2]
                    b_s = b_v[r, sl]
                    b_c = b_v[r, sl2]
                    pos = jnp.where(
                        even,
                        a_s * b_c + a_c * b_s,
                        a_c * b_c - a_s * b_s,
                    )
                    for b in range(batch):
                        plsc.addupdate(rows_b[s].at[b * CP + r, sl], pos)
                return carry

            lax.fori_loop(0, CP, row_body, 0)

            pending_out[j] = [
                pltpu.async_copy(
                    rows_b[s].at[pl.ds(b * CP, CP)],
                    out_hbm.at[pl.ds(b * seq + q0 + j * CP, CP)],
                    sem_o[s],
                )
                for b in range(batch)
            ]
        for j in (nch - 2, nch - 1):
            for c in pending_out[j]:
                c.wait()

    return emb_kernel


@jax.jit
def kernel(x, tok_table):
    batch, seq = x.shape
    a_tab, b_tab = _trig_tables(seq, D_MODEL)
    sc = _make_sc_kernel(batch, seq, D_MODEL)
    out = sc(x.astype(jnp.int32), tok_table, a_tab, b_tab)
    return out.reshape(batch, seq, D_MODEL)


# speedup vs baseline: 1.3844x; 1.3844x over previous
import functools

import jax
import jax.numpy as jnp
import numpy as np
from jax import lax
from jax.experimental import pallas as pl
from jax.experimental.pallas import tpu as pltpu
from jax.experimental.pallas import tpu_sc as plsc

VOCAB = 100000
D_MODEL = 768
MAX_LEN = 4096
LANES = 16

NUM_CORES = 2
NUM_SUBCORES = 16
NW = NUM_CORES * NUM_SUBCORES

CP = 16
NBUF = 4
LOOK = 2


@functools.lru_cache(maxsize=None)
def _sinusoidal_encoding(max_len, d_model):
    pos = np.arange(max_len, dtype=np.float32)[:, None]
    i2 = np.arange(0, d_model, 2, dtype=np.float32)
    div = np.power(np.float32(10000.0), (i2 / np.float32(d_model)).astype(np.float32))
    enc = np.zeros((max_len, d_model), dtype=np.float32)
    enc[:, 0::2] = np.sin((pos / div).astype(np.float32))
    enc[:, 1::2] = np.cos((pos / div).astype(np.float32))
    return jnp.asarray(enc)


def _make_sc_kernel(batch, seq, d_model):
    ppw = seq // NW
    nch = ppw // CP
    nunit = nch * batch
    mesh = plsc.VectorSubcoreMesh(core_axis_name="c", subcore_axis_name="s")

    @functools.partial(
        pl.kernel,
        out_type=jax.ShapeDtypeStruct((batch * seq, d_model), jnp.float32),
        mesh=mesh,
        scratch_types=(
            [pltpu.VMEM((batch, ppw), jnp.int32)]
            + [pltpu.VMEM((CP, d_model), jnp.float32) for _ in range(NBUF)]
            + [pltpu.VMEM((CP, d_model), jnp.float32) for _ in range(2)]
            + [pltpu.SemaphoreType.DMA for _ in range(2 * NBUF + 3)]
        ),
    )
    def emb_kernel(x_hbm, table_hbm, pos_hbm, out_hbm, idx_v, *bufs):
        rows_b = bufs[0:NBUF]
        pos_b = bufs[NBUF:NBUF + 2]
        sem_i = bufs[NBUF + 2]
        sem_p = bufs[NBUF + 3:NBUF + 5]
        sem_g = bufs[NBUF + 5:NBUF + 5 + NBUF]
        sem_o = bufs[NBUF + 5 + NBUF:NBUF + 5 + 2 * NBUF]

        wid = lax.axis_index("s") * NUM_CORES + lax.axis_index("c")
        q0 = wid * ppw

        idx_copies = [
            pltpu.async_copy(x_hbm.at[b, pl.ds(q0, ppw)], idx_v.at[b], sem_i)
            for b in range(batch)
        ]
        for c in idx_copies:
            c.wait()

        def start(u):
            j, b = divmod(u, batch)
            s = u % NBUF
            g = pltpu.async_copy(
                table_hbm.at[idx_v.at[b, pl.ds(j * CP, CP)]],
                rows_b[s],
                sem_g[s],
            )
            p = None
            if b == 0:
                p = pltpu.async_copy(
                    pos_hbm.at[pl.ds(q0 + j * CP, CP)],
                    pos_b[j % 2],
                    sem_p[j % 2],
                )
            return g, p

        started = {u: start(u) for u in range(LOOK)}
        pending_out = {}
        pos_waited = set()
        for u in range(nunit):
            j, b = divmod(u, batch)
            s = u % NBUF
            v = u + LOOK
            if v < nunit:
                if v - NBUF >= 0:
                    pending_out[v - NBUF].wait()
                started[v] = start(v)
            g, p = started[u]
            g.wait()
            if j not in pos_waited:
                pos_waited.add(j)
                gp = started[j * batch][1]
                gp.wait()

            def row_body(r, carry, s=s, pj=j % 2):
                for k in range(d_model // LANES):
                    sl = pl.ds(k * LANES, LANES)
                    plsc.addupdate(rows_b[s].at[r, sl], pos_b[pj][r, sl])
                return carry

            lax.fori_loop(0, CP, row_body, 0)

            pending_out[u] = pltpu.async_copy(
                rows_b[s],
                out_hbm.at[pl.ds(b * seq + q0 + j * CP, CP)],
                sem_o[s],
            )
        for u in range(max(0, nunit - NBUF), nunit):
            pending_out[u].wait()

    return emb_kernel


@jax.jit
def kernel(x, tok_table):
    batch, seq = x.shape
    pos = _sinusoidal_encoding(MAX_LEN, D_MODEL)[:seq, :]
    sc = _make_sc_kernel(batch, seq, D_MODEL)
    out = sc(x.astype(jnp.int32), tok_table, pos)
    return out.reshape(batch, seq, D_MODEL)


# speedup vs baseline: 1.4624x; 1.0564x over previous
import functools

import jax
import jax.numpy as jnp
import numpy as np
from jax import lax
from jax.experimental import pallas as pl
from jax.experimental.pallas import tpu as pltpu
from jax.experimental.pallas import tpu_sc as plsc

VOCAB = 100000
D_MODEL = 768
MAX_LEN = 4096
LANES = 16

NUM_CORES = 2
NUM_SUBCORES = 16
NW = NUM_CORES * NUM_SUBCORES

CP = 16


@functools.lru_cache(maxsize=None)
def _sinusoidal_encoding(max_len, d_model):
    pos = np.arange(max_len, dtype=np.float32)[:, None]
    i2 = np.arange(0, d_model, 2, dtype=np.float32)
    div = np.power(np.float32(10000.0), (i2 / np.float32(d_model)).astype(np.float32))
    enc = np.zeros((max_len, d_model), dtype=np.float32)
    enc[:, 0::2] = np.sin((pos / div).astype(np.float32))
    enc[:, 1::2] = np.cos((pos / div).astype(np.float32))
    return jnp.asarray(enc)


def _make_sc_kernel(batch, seq, d_model):
    ppw = seq // NW
    nch = ppw // CP
    rows = batch * CP
    mesh = plsc.VectorSubcoreMesh(core_axis_name="c", subcore_axis_name="s")

    @functools.partial(
        pl.kernel,
        out_type=jax.ShapeDtypeStruct((batch * seq, d_model), jnp.float32),
        mesh=mesh,
        scratch_types=[
            pltpu.VMEM((nch, rows), jnp.int32),
            pltpu.VMEM((rows, d_model), jnp.float32),
            pltpu.VMEM((rows, d_model), jnp.float32),
            pltpu.VMEM((CP, d_model), jnp.float32),
            pltpu.VMEM((CP, d_model), jnp.float32),
            pltpu.SemaphoreType.DMA,
            pltpu.SemaphoreType.DMA,
            pltpu.SemaphoreType.DMA,
            pltpu.SemaphoreType.DMA,
            pltpu.SemaphoreType.DMA,
            pltpu.SemaphoreType.DMA,
            pltpu.SemaphoreType.DMA,
        ],
    )
    def emb_kernel(x_hbm, table_hbm, pos_hbm, out_hbm,
                   idx_v, rows0, rows1, pos0, pos1,
                   sem_i, sem_g0, sem_g1, sem_p0, sem_p1, sem_o0, sem_o1):
        rows_b = (rows0, rows1)
        pos_b = (pos0, pos1)
        sem_g = (sem_g0, sem_g1)
        sem_p = (sem_p0, sem_p1)
        sem_o = (sem_o0, sem_o1)

        wid = lax.axis_index("s") * NUM_CORES + lax.axis_index("c")
        q0 = wid * ppw

        idx_copies = [
            pltpu.async_copy(
                x_hbm.at[b, pl.ds(q0 + j * CP, CP)],
                idx_v.at[j, pl.ds(b * CP, CP)],
                sem_i,
            )
            for j in range(nch)
            for b in range(batch)
        ]
        for c in idx_copies:
            c.wait()

        def start(j):
            s = j % 2
            gs = [
                pltpu.async_copy(
                    table_hbm.at[idx_v.at[j]],
                    rows_b[s],
                    sem_g[s],
                )
            ]
            ps = pltpu.async_copy(
                pos_hbm.at[pl.ds(q0 + j * CP, CP)], pos_b[s], sem_p[s]
            )
            return gs, ps

        started = {0: start(0)}
        pending_out = {}
        for j in range(nch):
            s = j % 2
            if j + 1 < nch:
                if j - 1 >= 0:
                    for c in pending_out[j - 1]:
                        c.wait()
                started[j + 1] = start(j + 1)
            gs, ps = started[j]
            for c in gs:
                c.wait()
            ps.wait()

            def row_body(r, carry, s=s):
                for k in range(d_model // LANES):
                    sl = pl.ds(k * LANES, LANES)
                    pos = pos_b[s][r, sl]
                    for b in range(batch):
                        plsc.addupdate(rows_b[s].at[b * CP + r, sl], pos)
                return carry

            lax.fori_loop(0, CP, row_body, 0)

            pending_out[j] = [
                pltpu.async_copy(
                    rows_b[s].at[pl.ds(b * CP, CP)],
                    out_hbm.at[pl.ds(b * seq + q0 + j * CP, CP)],
                    sem_o[s],
                )
                for b in range(batch)
            ]
        for j in (nch - 2, nch - 1):
            for c in pending_out[j]:
                c.wait()

    return emb_kernel


@jax.jit
def kernel(x, tok_table):
    batch, seq = x.shape
    pos = _sinusoidal_encoding(MAX_LEN, D_MODEL)[:seq, :]
    sc = _make_sc_kernel(batch, seq, D_MODEL)
    out = sc(x.astype(jnp.int32), tok_table, pos)
    return out.reshape(batch, seq, D_MODEL)


# speedup vs baseline: 1.5204x; 1.0397x over previous
import functools

import jax
import jax.numpy as jnp
import numpy as np
from jax import lax
from jax.experimental import pallas as pl
from jax.experimental.pallas import tpu as pltpu
from jax.experimental.pallas import tpu_sc as plsc

VOCAB = 100000
D_MODEL = 768
MAX_LEN = 4096
LANES = 16

NUM_CORES = 2
NUM_SUBCORES = 16
NW = NUM_CORES * NUM_SUBCORES

CP = 16


@functools.lru_cache(maxsize=None)
def _sinusoidal_encoding(max_len, d_model):
    pos = np.arange(max_len, dtype=np.float32)[:, None]
    i2 = np.arange(0, d_model, 2, dtype=np.float32)
    div = np.power(np.float32(10000.0), (i2 / np.float32(d_model)).astype(np.float32))
    enc = np.zeros((max_len, d_model), dtype=np.float32)
    enc[:, 0::2] = np.sin((pos / div).astype(np.float32))
    enc[:, 1::2] = np.cos((pos / div).astype(np.float32))
    return jnp.asarray(enc)


NBUF = 4
LOOK = 2
BPU = 2


def _make_sc_kernel(batch, seq, d_model):
    ppw = seq // NW
    nch = ppw // CP
    upc = batch // BPU
    nunit = nch * upc
    urows = BPU * CP
    mesh = plsc.VectorSubcoreMesh(core_axis_name="c", subcore_axis_name="s")

    @functools.partial(
        pl.kernel,
        out_type=jax.ShapeDtypeStruct((batch * seq, d_model), jnp.float32),
        mesh=mesh,
        scratch_types=(
            [pltpu.VMEM((nch, batch * CP), jnp.int32)]
            + [pltpu.VMEM((urows, d_model), jnp.float32) for _ in range(NBUF)]
            + [pltpu.VMEM((CP, d_model), jnp.float32) for _ in range(2)]
            + [pltpu.SemaphoreType.DMA for _ in range(2 * NBUF + 3)]
        ),
    )
    def emb_kernel(x_hbm, table_hbm, pos_hbm, out_hbm, idx_v, *bufs):
        rows_b = bufs[0:NBUF]
        pos_b = bufs[NBUF:NBUF + 2]
        sem_i = bufs[NBUF + 2]
        sem_p = bufs[NBUF + 3:NBUF + 5]
        sem_g = bufs[NBUF + 5:NBUF + 5 + NBUF]
        sem_o = bufs[NBUF + 5 + NBUF:NBUF + 5 + 2 * NBUF]

        wid = lax.axis_index("s") * NUM_CORES + lax.axis_index("c")
        q0 = wid * ppw

        idx_copies = [
            pltpu.async_copy(
                x_hbm.at[b, pl.ds(q0 + j * CP, CP)],
                idx_v.at[j, pl.ds(b * CP, CP)],
                sem_i,
            )
            for j in range(nch)
            for b in range(batch)
        ]
        for c in idx_copies:
            c.wait()

        def start(u):
            j, h = divmod(u, upc)
            s = u % NBUF
            g = pltpu.async_copy(
                table_hbm.at[idx_v.at[j, pl.ds(h * urows, urows)]],
                rows_b[s],
                sem_g[s],
            )
            p = None
            if h == 0:
                p = pltpu.async_copy(
                    pos_hbm.at[pl.ds(q0 + j * CP, CP)],
                    pos_b[j % 2],
                    sem_p[j % 2],
                )
            return g, p

        started = {u: start(u) for u in range(LOOK)}
        pending_out = {}
        pos_waited = set()
        for u in range(nunit):
            j, h = divmod(u, upc)
            s = u % NBUF
            v = u + LOOK
            if v < nunit:
                if v - NBUF >= 0:
                    for c in pending_out[v - NBUF]:
                        c.wait()
                started[v] = start(v)
            g, _ = started[u]
            g.wait()
            if j not in pos_waited:
                pos_waited.add(j)
                started[j * upc][1].wait()

            def row_body(r, carry, s=s, pj=j % 2):
                for k in range(d_model // LANES):
                    sl = pl.ds(k * LANES, LANES)
                    pos = pos_b[pj][r, sl]
                    for i in range(BPU):
                        plsc.addupdate(rows_b[s].at[i * CP + r, sl], pos)
                return carry

            lax.fori_loop(0, CP, row_body, 0)

            pending_out[u] = [
                pltpu.async_copy(
                    rows_b[s].at[pl.ds(i * CP, CP)],
                    out_hbm.at[pl.ds((h * BPU + i) * seq + q0 + j * CP, CP)],
                    sem_o[s],
                )
                for i in range(BPU)
            ]
        for u in range(max(0, nunit - NBUF), nunit):
            for c in pending_out[u]:
                c.wait()

    return emb_kernel


@jax.jit
def kernel(x, tok_table):
    batch, seq = x.shape
    pos = _sinusoidal_encoding(MAX_LEN, D_MODEL)[:seq, :]
    sc = _make_sc_kernel(batch, seq, D_MODEL)
    out = sc(x.astype(jnp.int32), tok_table, pos)
    return out.reshape(batch, seq, D_MODEL)
